# baseline (speedup 1.0000x reference)
import functools

import jax
import jax.numpy as jnp
from jax import lax
from jax.experimental import pallas as pl
from jax.experimental.pallas import tpu as pltpu
from jax.experimental.pallas import tpu_sc as plsc

VOCAB_SIZE = 1000000
EMBED_DIM = 64
SEQ_LEN = 200
BATCH = 1024

NUM_WORKERS = 32
LANES = 16

CHUNK_T = 128
VOCAB_PAD = 1000064
NCHUNKS = VOCAB_PAD // CHUNK_T
CPW = (NCHUNKS + NUM_WORKERS - 1) // NUM_WORKERS
CPW_MAIN = CPW - 1

BLK = 128
NBLK = BATCH // BLK
UNITS = SEQ_LEN * NBLK
UPW = UNITS // NUM_WORKERS
GROUPS = BLK // LANES


def _transpose_chunk(c_ref, d_ref, row_idxs):
    def j_body(j, carry):
        cols = [jnp.broadcast_to(2 * j, (LANES,)),
                jnp.broadcast_to(2 * j + 1, (LANES,))]
        for c0h in range(2 * EMBED_DIM // LANES):
            vals = plsc.load_gather(
                c_ref, [row_idxs[c0h % 4], cols[c0h // 4]])
            d_ref[j, pl.ds(c0h * LANES, LANES)] = vals
        return carry

    lax.fori_loop(0, CHUNK_T // 2, j_body, 0, unroll=8)


def _repack_body(tokT_hbm, dense_hbm, c_a, c_b, d_a, d_b,
                 gsem_a, gsem_b, ssem_a, ssem_b):
    wid = lax.axis_index("s") * 2 + lax.axis_index("c")
    c_v = (c_a, c_b)
    d_v = (d_a, d_b)
    gsem = (gsem_a, gsem_b)
    ssem = (ssem_a, ssem_b)

    row_idxs = [lax.iota(jnp.int32, LANES) + q * LANES for q in range(4)]

    def t_of(m):
        return pl.multiple_of((wid + NUM_WORKERS * m) * CHUNK_T, CHUNK_T)

    def c_dst(b):
        return c_v[b].at[:, pl.ds(0, CHUNK_T)]

    for b in range(2):
        pltpu.async_copy(tokT_hbm.at[:, pl.ds(t_of(b), CHUNK_T)],
                         c_dst(b), gsem[b])

    def body(i, carry):
        for b in range(2):
            m = 2 * i + b
            t0 = t_of(m)
            pltpu.make_async_copy(
                tokT_hbm.at[:, pl.ds(0, CHUNK_T)], c_dst(b), gsem[b]).wait()

            @pl.when(i > 0)
            def _():
                pltpu.make_async_copy(
                    d_v[b], dense_hbm.at[pl.ds(0, CHUNK_T // 2)],
                    ssem[b]).wait()

            _transpose_chunk(c_v[b], d_v[b], row_idxs)

            @pl.when(m + 2 < CPW_MAIN)
            def _():
                pltpu.async_copy(
                    tokT_hbm.at[:, pl.ds(t_of(m + 2), CHUNK_T)],
                    c_dst(b), gsem[b])

            d0 = pl.multiple_of(t0 // 2, CHUNK_T // 2)
            pltpu.async_copy(d_v[b], dense_hbm.at[pl.ds(d0, CHUNK_T // 2)],
                             ssem[b])
        return carry

    lax.fori_loop(0, CPW_MAIN // 2, body, 0)
    for b in range(2):
        pltpu.make_async_copy(
            d_v[b], dense_hbm.at[pl.ds(0, CHUNK_T // 2)], ssem[b]).wait()

    @pl.when(wid + NUM_WORKERS * CPW_MAIN < NCHUNKS)
    def _():
        t0 = t_of(CPW_MAIN)
        pltpu.sync_copy(tokT_hbm.at[:, pl.ds(t0, CHUNK_T)],
                        c_a.at[:, pl.ds(0, CHUNK_T)])
        _transpose_chunk(c_a, d_a, row_idxs)
        d0 = pl.multiple_of(t0 // 2, CHUNK_T // 2)
        pltpu.sync_copy(d_a, dense_hbm.at[pl.ds(d0, CHUNK_T // 2)])


def _gather_unit_compute(idx_ref, g_ref, m_ref, pos_v, s, row_idxs):
    col_bases = [(idx_ref[pl.ds(c0 * LANES, LANES)] & 1) * EMBED_DIM
                 for c0 in range(GROUPS)]

    def a_body(a, carry):
        pvec = plsc.load_gather(
            pos_v, [jnp.broadcast_to(s * EMBED_DIM + a, (LANES,))])
        for c0 in range(GROUPS):
            vals = plsc.load_gather(g_ref, [row_idxs[c0], col_bases[c0] + a])
            m_ref[a, pl.ds(c0 * LANES, LANES)] = vals + pvec
        return carry

    lax.fori_loop(0, EMBED_DIM, a_body, 0, unroll=4)


def _gather_body(xT_hbm, tok_hbm, pos_hbm, out_hbm,
                 idx_a, idx_b, rid_a, rid_b, g_a, g_b, m_a, m_b, pos_v,
                 gsem_a, gsem_b, ssem_a, ssem_b):
    wid = lax.axis_index("s") * 2 + lax.axis_index("c")
    t0 = wid * UPW
    idx_v = (idx_a, idx_b)
    rid_v = (rid_a, rid_b)
    g_v = (g_a, g_b)
    m_v = (m_a, m_b)
    gsem = (gsem_a, gsem_b)
    ssem = (ssem_a, ssem_b)

    pltpu.sync_copy(pos_hbm, pos_v)

    row_idxs = [lax.iota(jnp.int32, LANES) + c0 * LANES
                for c0 in range(GROUPS)]

    def g_dst(b):
        return g_v[b].at[:, pl.ds(0, 2 * EMBED_DIM)]

    def launch(t, b):
        s = t // NBLK
        v = t % NBLK
        pltpu.sync_copy(xT_hbm.at[s, pl.ds(v * BLK, BLK)], idx_v[b])
        for i in range(GROUPS):
            sl = pl.ds(i * LANES, LANES)
            rid_v[b][sl] = lax.shift_right_logical(idx_v[b][sl], 1)
        pltpu.async_copy(tok_hbm.at[rid_v[b]], g_dst(b), gsem[b])

    for b in range(2):
        launch(t0 + b, b)

    def body(j, carry):
        for b in range(2):
            t = t0 + 2 * j + b
            s = t // NBLK
            v = t % NBLK
            pltpu.make_async_copy(
                tok_hbm.at[rid_v[b]], g_dst(b), gsem[b]).wait()

            @pl.when(j > 0)
            def _():
                pltpu.make_async_copy(
                    m_v[b], out_hbm.at[0, :, pl.ds(0, BLK)], ssem[b]).wait()

            _gather_unit_compute(idx_v[b], g_v[b], m_v[b], pos_v, s,
                                 row_idxs)

            @pl.when(2 * j + b + 2 < UPW)
            def _():
                launch(t + 2, b)

            pltpu.async_copy(m_v[b], out_hbm.at[s, :, pl.ds(v * BLK, BLK)],
                             ssem[b])
        return carry

    lax.fori_loop(0, UPW // 2, body, 0)
    for b in range(2):
        pltpu.make_async_copy(
            m_v[b], out_hbm.at[0, :, pl.ds(0, BLK)], ssem[b]).wait()


def kernel(x, token_table, pos_table):
    xT = x.T.astype(jnp.int32)
    tokT = token_table.T
    pos_flat = pos_table.reshape(SEQ_LEN * EMBED_DIM)
    mesh = plsc.VectorSubcoreMesh(core_axis_name="c", subcore_axis_name="s")

    repack = functools.partial(
        pl.kernel,
        mesh=mesh,
        out_type=jax.ShapeDtypeStruct((VOCAB_PAD // 2, 2 * EMBED_DIM),
                                      jnp.float32),
        scratch_types=[
            pltpu.VMEM((EMBED_DIM, CHUNK_T + 1), jnp.float32),
            pltpu.VMEM((EMBED_DIM, CHUNK_T + 1), jnp.float32),
            pltpu.VMEM((CHUNK_T // 2, 2 * EMBED_DIM), jnp.float32),
            pltpu.VMEM((CHUNK_T // 2, 2 * EMBED_DIM), jnp.float32),
            pltpu.SemaphoreType.DMA,
            pltpu.SemaphoreType.DMA,
            pltpu.SemaphoreType.DMA,
            pltpu.SemaphoreType.DMA,
        ],
        compiler_params=pltpu.CompilerParams(
            needs_layout_passes=False, disable_bounds_checks=True),
    )(_repack_body)
    dense = repack(tokT)

    gather = functools.partial(
        pl.kernel,
        mesh=mesh,
        out_type=jax.ShapeDtypeStruct((SEQ_LEN, EMBED_DIM, BATCH),
                                      jnp.float32),
        scratch_types=[
            pltpu.VMEM((BLK,), jnp.int32),
            pltpu.VMEM((BLK,), jnp.int32),
            pltpu.VMEM((BLK,), jnp.int32),
            pltpu.VMEM((BLK,), jnp.int32),
            pltpu.VMEM((BLK, 2 * EMBED_DIM + 1), jnp.float32),
            pltpu.VMEM((BLK, 2 * EMBED_DIM + 1), jnp.float32),
            pltpu.VMEM((EMBED_DIM, BLK), jnp.float32),
            pltpu.VMEM((EMBED_DIM, BLK), jnp.float32),
            pltpu.VMEM((SEQ_LEN * EMBED_DIM,), jnp.float32),
            pltpu.SemaphoreType.DMA,
            pltpu.SemaphoreType.DMA,
            pltpu.SemaphoreType.DMA,
            pltpu.SemaphoreType.DMA,
        ],
        compiler_params=pltpu.CompilerParams(needs_layout_passes=False),
    )(_gather_body)
    out = gather(xT, dense, pos_flat)
    return out.transpose(2, 0, 1)


# speedup vs baseline: 2.2868x; 2.2868x over previous
import functools

import jax
import jax.numpy as jnp
from jax import lax
from jax.experimental import pallas as pl
from jax.experimental.pallas import tpu as pltpu
from jax.experimental.pallas import tpu_sc as plsc

VOCAB_SIZE = 1000000
EMBED_DIM = 64
SEQ_LEN = 200
BATCH = 1024

ROWS = BATCH * SEQ_LEN
NUM_WORKERS = 32
RPW = ROWS // NUM_WORKERS
CHUNK = SEQ_LEN
NUM_CHUNKS = RPW // CHUNK
LANES = 16
VPR = EMBED_DIM // LANES
WIDE = 2 * EMBED_DIM


def _body(x_hbm, tok_hbm, pos_hbm, out_hbm,
          idx_a, idx_b, buf_a, buf_b, pos_v,
          gsem_a, gsem_b, ssem_a, ssem_b):
    wid = lax.axis_index("s") * 2 + lax.axis_index("c")
    base0 = wid * RPW

    idx_v = (idx_a, idx_b)
    buf_v = (buf_a, buf_b)
    gsem = (gsem_a, gsem_b)
    ssem = (ssem_a, ssem_b)

    pltpu.sync_copy(pos_hbm, pos_v)

    def launch(c, b):
        pltpu.sync_copy(x_hbm.at[pl.ds(base0 + c * CHUNK, CHUNK)], idx_v[b])
        pltpu.async_copy(tok_hbm.at[idx_v[b]], buf_v[b], gsem[b])

    for b in range(2):
        launch(b, b)

    def body(j, carry):
        for b in range(2):
            c = 2 * j + b
            pltpu.make_async_copy(
                tok_hbm.at[idx_v[b]], buf_v[b], gsem[b]).wait()

            @pl.when(j > 0)
            def _():
                pltpu.make_async_copy(
                    buf_v[b], out_hbm.at[pl.ds(0, CHUNK)], ssem[b]).wait()

            def row_body(r, carry2):
                for jj in range(VPR):
                    sl = pl.ds(jj * LANES, LANES)
                    plsc.addupdate(buf_v[b].at[r, sl], pos_v[r, sl])
                return carry2

            lax.fori_loop(0, CHUNK, row_body, 0, unroll=4)

            pltpu.async_copy(
                buf_v[b], out_hbm.at[pl.ds(base0 + c * CHUNK, CHUNK)],
                ssem[b])

            @pl.when(c + 2 < NUM_CHUNKS)
            def _():
                launch(c + 2, b)
        return carry

    lax.fori_loop(0, NUM_CHUNKS // 2, body, 0)
    for b in range(2):
        pltpu.make_async_copy(
            buf_v[b], out_hbm.at[pl.ds(0, CHUNK)], ssem[b]).wait()


def kernel(x, token_table, pos_table):
    xf = x.reshape(ROWS).astype(jnp.int32)
    tok2 = jnp.concatenate([token_table, token_table], axis=1)
    mesh = plsc.VectorSubcoreMesh(core_axis_name="c", subcore_axis_name="s")
    run = functools.partial(
        pl.kernel,
        mesh=mesh,
        out_type=jax.ShapeDtypeStruct((ROWS, WIDE), jnp.float32),
        scratch_types=[
            pltpu.VMEM((CHUNK,), jnp.int32),
            pltpu.VMEM((CHUNK,), jnp.int32),
            pltpu.VMEM((CHUNK, WIDE), jnp.float32),
            pltpu.VMEM((CHUNK, WIDE), jnp.float32),
            pltpu.VMEM((CHUNK, EMBED_DIM), jnp.float32),
            pltpu.SemaphoreType.DMA,
            pltpu.SemaphoreType.DMA,
            pltpu.SemaphoreType.DMA,
            pltpu.SemaphoreType.DMA,
        ],
        compiler_params=pltpu.CompilerParams(needs_layout_passes=False),
    )(_body)
    out = run(xf, tok2, pos_table)
    return out[:, :EMBED_DIM].reshape(BATCH, SEQ_LEN, EMBED_DIM)


# speedup vs baseline: 2.3740x; 1.0381x over previous
import functools

import jax
import jax.numpy as jnp
from jax import lax
from jax.experimental import pallas as pl
from jax.experimental.pallas import tpu as pltpu
from jax.experimental.pallas import tpu_sc as plsc

VOCAB_SIZE = 1000000
EMBED_DIM = 64
SEQ_LEN = 200
BATCH = 1024

ROWS = BATCH * SEQ_LEN
NUM_WORKERS = 32
ROWS_PER_WORKER = ROWS // NUM_WORKERS
SEQ_PER_CHUNK = 2
CHUNK = SEQ_PER_CHUNK * SEQ_LEN
NUM_CHUNKS = ROWS_PER_WORKER // CHUNK
LANES = 16
VPR = EMBED_DIM // LANES


def _body(x_hbm, tok_hbm, pos_hbm, out_hbm,
          idx_a, idx_b, buf_a, buf_b, pos_v,
          gsem_a, gsem_b, ssem_a, ssem_b):
    wid = lax.axis_index("s") * 2 + lax.axis_index("c")
    base0 = wid * ROWS_PER_WORKER

    idx_v = (idx_a, idx_b)
    buf_v = (buf_a, buf_b)
    gsem = (gsem_a, gsem_b)
    ssem = (ssem_a, ssem_b)

    pltpu.sync_copy(pos_hbm, pos_v)

    gather = [None, None]
    store = [None, None]

    def launch(c):
        b = c % 2
        pltpu.sync_copy(x_hbm.at[pl.ds(base0 + c * CHUNK, CHUNK)], idx_v[b])
        gather[b] = pltpu.async_copy(tok_hbm.at[idx_v[b]], buf_v[b], gsem[b])

    launch(0)
    for c in range(NUM_CHUNKS):
        b = c % 2
        nb = (c + 1) % 2
        if c + 1 < NUM_CHUNKS:
            if store[nb] is not None:
                store[nb].wait()
            launch(c + 1)
        gather[b].wait()

        def row_body(r, carry):
            for j in range(VPR):
                sl = pl.ds(j * LANES, LANES)
                plsc.addupdate(buf_v[b].at[r, sl], pos_v[r, sl])
            return carry

        lax.fori_loop(0, CHUNK, row_body, 0, unroll=4)
        store[b] = pltpu.async_copy(
            buf_v[b], out_hbm.at[pl.ds(base0 + c * CHUNK, CHUNK)], ssem[b])

    store[(NUM_CHUNKS - 1) % 2].wait()
    store[NUM_CHUNKS % 2].wait()


def kernel(x, token_table, pos_table):
    xf = x.reshape(ROWS).astype(jnp.int32)
    pos_tiled = jnp.tile(pos_table, (SEQ_PER_CHUNK, 1))
    mesh = plsc.VectorSubcoreMesh(core_axis_name="c", subcore_axis_name="s")
    run = functools.partial(
        pl.kernel,
        mesh=mesh,
        out_type=jax.ShapeDtypeStruct((ROWS, EMBED_DIM), jnp.float32),
        scratch_types=[
            pltpu.VMEM((CHUNK,), jnp.int32),
            pltpu.VMEM((CHUNK,), jnp.int32),
            pltpu.VMEM((CHUNK, EMBED_DIM), jnp.float32),
            pltpu.VMEM((CHUNK, EMBED_DIM), jnp.float32),
            pltpu.VMEM((CHUNK, EMBED_DIM), jnp.float32),
            pltpu.SemaphoreType.DMA,
            pltpu.SemaphoreType.DMA,
            pltpu.SemaphoreType.DMA,
            pltpu.SemaphoreType.DMA,
        ],
        compiler_params=pltpu.CompilerParams(use_tc_tiling_on_sc=False),
    )(_body)
    out = run(xf, token_table, pos_tiled)
    return out.reshape(BATCH, SEQ_LEN, EMBED_DIM)
